# baseline (speedup 1.0000x reference)
import functools

import jax
import jax.numpy as jnp
from jax import lax
from jax.experimental import pallas as pl
from jax.experimental.pallas import tpu as pltpu
from jax.experimental.pallas import tpu_sc as plsc

NC = 2
NS = 16
LANES = 16
NW = NC * NS

VOCAB = 1000000
BATCH = 16384
DIM = 64
NNEG = 20
KD = DIM // LANES

BPW = BATCH // NW
CHUNK = 64
NCHUNK = BPW // CHUNK
SLEN = 128
NNEGC = CHUNK * NNEG
NSTREAM = NNEGC // SLEN

PCOLS = 32000
NFULL = 31
TCOLS = VOCAB - NFULL * PCOLS
MAIN_END = NFULL * PCOLS
PGRID = NFULL + 1


def _pack_body(xt_hbm, out_hbm, buf0, buf1, ob0, ob1, tbuf, tob,
               si0, si1, so0, so1):
    i = pl.program_id(0)

    def start_in(blk, buf, sem):
        pltpu.make_async_copy(
            xt_hbm.at[:, pl.ds(blk * PCOLS, PCOLS)], buf, sem).start()

    even = lax.rem(i, 2) == 0

    @pl.when(i == 0)
    def _():
        start_in(0, buf0, si0)

    @pl.when(jnp.logical_and(i + 1 < NFULL, even))
    def _():
        start_in(i + 1, buf1, si1)

    @pl.when(jnp.logical_and(i + 1 < NFULL, jnp.logical_not(even)))
    def _():
        start_in(i + 1, buf0, si0)

    def pair_rows(x, n):
        xt = jax.lax.dot(x.T, jnp.eye(DIM, dtype=jnp.float32),
                         preferred_element_type=jnp.float32)
        q = n // 4
        parts = []
        for h in range(4):
            seg = lax.slice(xt, (h * q, 0), ((h + 1) * q, DIM))
            u = lax.bitcast_convert_type(seg, jnp.uint32)
            r = (u + jnp.uint32(0x7FFF) + ((u >> 16) & jnp.uint32(1))) >> 16
            lo = lax.slice(r, (0, 0), (q, DIM // 2))
            hi = lax.slice(r, (0, DIM // 2), (q, DIM))
            parts.append(lo | (hi << 16))
        return lax.bitcast_convert_type(jnp.concatenate(parts, axis=1),
                                        jnp.int32)

    def do_block(buf, ob, sin, sout):
        pltpu.make_async_copy(xt_hbm.at[:, pl.ds(0, PCOLS)], buf, sin).wait()
        ob[...] = pair_rows(buf[...], PCOLS)
        pltpu.make_async_copy(
            ob, out_hbm.at[pl.ds(i * (PCOLS // 4), PCOLS // 4), :],
            sout).start()

    @pl.when(jnp.logical_and(i < NFULL, even))
    def _():
        @pl.when(i >= 2)
        def _():
            pltpu.make_async_copy(
                ob0, out_hbm.at[pl.ds(0, PCOLS // 4), :], so0).wait()
        do_block(buf0, ob0, si0, so0)

    @pl.when(jnp.logical_and(i < NFULL, jnp.logical_not(even)))
    def _():
        @pl.when(i >= 3)
        def _():
            pltpu.make_async_copy(
                ob1, out_hbm.at[pl.ds(0, PCOLS // 4), :], so1).wait()
        do_block(buf1, ob1, si1, so1)

    @pl.when(i == NFULL)
    def _():
        pltpu.make_async_copy(
            xt_hbm.at[:, pl.ds(NFULL * PCOLS, TCOLS)], tbuf, si0).start()
        pltpu.make_async_copy(
            xt_hbm.at[:, pl.ds(NFULL * PCOLS, TCOLS)], tbuf, si0).wait()
        tob[...] = pair_rows(tbuf[...], TCOLS)
        pltpu.make_async_copy(
            tob, out_hbm.at[pl.ds(NFULL * PCOLS // 4, TCOLS // 4), :],
            so0).start()
        pltpu.make_async_copy(
            ob1, out_hbm.at[pl.ds(0, PCOLS // 4), :], so1).wait()
        pltpu.make_async_copy(
            ob0, out_hbm.at[pl.ds(0, PCOLS // 4), :], so0).wait()
        pltpu.make_async_copy(
            tob, out_hbm.at[pl.ds(0, TCOLS // 4), :], so0).wait()


_pack = pl.pallas_call(
    _pack_body,
    grid=(PGRID,),
    in_specs=[pl.BlockSpec(memory_space=pl.ANY)],
    out_specs=pl.BlockSpec(memory_space=pl.ANY),
    out_shape=jax.ShapeDtypeStruct((VOCAB // 4, 2 * DIM), jnp.int32),
    compiler_params=pltpu.CompilerParams(fuse_transposed_lhs_in_matmul=True),
    scratch_shapes=[pltpu.VMEM((DIM, PCOLS), jnp.float32),
                    pltpu.VMEM((DIM, PCOLS), jnp.float32),
                    pltpu.VMEM((PCOLS // 4, 2 * DIM), jnp.int32),
                    pltpu.VMEM((PCOLS // 4, 2 * DIM), jnp.int32),
                    pltpu.VMEM((DIM, TCOLS), jnp.float32),
                    pltpu.VMEM((TCOLS // 4, 2 * DIM), jnp.int32),
                    pltpu.SemaphoreType.DMA,
                    pltpu.SemaphoreType.DMA,
                    pltpu.SemaphoreType.DMA,
                    pltpu.SemaphoreType.DMA],
)


def _sc_scores_body(Uemb, Vemb, upos, vpos, vnegf,
                    spos_out, sneg_out,
                    uidx, vidx, negidx, urows, vrows, negrows,
                    spos_acc, sneg_acc, sem):
    cid = lax.axis_index("c")
    sid = lax.axis_index("s")
    wid = cid * NS + sid
    wbase = wid * BPW

    def to_gidx(x):
        xt = x - MAIN_END
        blk = ((x >> 8) * 33555) >> 22
        rem = x - blk * PCOLS
        q = PCOLS // 4
        gm = blk * PCOLS + 4 * rem
        for t in (1, 2, 3):
            gm = gm - jnp.where(rem >= t * q, 4 * q - 1, 0)
        tq = TCOLS // 4
        gt = MAIN_END + 4 * xt
        for t in (1, 2, 3):
            gt = gt - jnp.where(xt >= t * tq, 4 * tq - 1, 0)
        return jnp.where(x >= MAIN_END, gt, gm)

    def chunk_body(ch, _):
        base = wbase + ch * CHUNK
        gchunk = wid * NCHUNK + ch
        pltpu.sync_copy(upos.at[pl.ds(base, CHUNK)], uidx)
        pltpu.sync_copy(vpos.at[pl.ds(base, CHUNK)], vidx)
        pltpu.sync_copy(vnegf.at[gchunk], negidx)
        for t in range(CHUNK // LANES):
            sl = pl.ds(t * LANES, LANES)
            uidx[sl] = to_gidx(uidx[sl])
            vidx[sl] = to_gidx(vidx[sl])
        for t in range(NNEGC // LANES):
            sl = pl.ds(t * LANES, LANES)
            negidx[sl] = to_gidx(negidx[sl])
        cps = [pltpu.async_copy(Uemb.at[uidx], urows, sem),
               pltpu.async_copy(Vemb.at[vidx], vrows, sem)]
        for s in range(NSTREAM):
            cps.append(pltpu.async_copy(
                Vemb.at[negidx.at[pl.ds(s * SLEN, SLEN)]],
                negrows.at[pl.ds(s * SLEN, SLEN), :], sem))
        for cp in cps:
            cp.wait()

        def load_row(ref, r):
            out = []
            for h in range(2):
                iv = ref[r, pl.ds(h * LANES, LANES)]
                lo = lax.bitcast_convert_type(iv << 16, jnp.float32)
                hi = lax.bitcast_convert_type(iv & jnp.int32(-65536),
                                              jnp.float32)
                out += [lo, hi]
            return out

        def item_body(b, _):
            u = load_row(urows, b)
            v = load_row(vrows, b)
            p = u[0] * v[0]
            for k in range(1, KD):
                p = p + u[k] * v[k]
            r0 = b * NNEG
            acc = load_row(negrows, r0)
            for j in range(1, NNEG):
                n = load_row(negrows, r0 + j)
                for k in range(KD):
                    acc[k] = acc[k] + n[k]
            q = acc[0] * u[0]
            for k in range(1, KD):
                q = q + acc[k] * u[k]
            off = (ch * CHUNK + b) * LANES
            spos_acc[pl.ds(off, LANES)] = p
            sneg_acc[pl.ds(off, LANES)] = q
            return 0

        lax.fori_loop(0, CHUNK, item_body, 0)
        return 0

    lax.fori_loop(0, NCHUNK, chunk_body, 0)
    pltpu.sync_copy(spos_acc, spos_out.at[pl.ds(wbase * LANES, BPW * LANES)])
    pltpu.sync_copy(sneg_acc, sneg_out.at[pl.ds(wbase * LANES, BPW * LANES)])


@functools.cache
def _sc_scores():
  return pl.kernel(
    _sc_scores_body,
    out_type=(jax.ShapeDtypeStruct((BATCH * LANES,), jnp.float32),
              jax.ShapeDtypeStruct((BATCH * LANES,), jnp.float32)),
    mesh=plsc.VectorSubcoreMesh(core_axis_name="c", subcore_axis_name="s",
                                num_cores=NC, num_subcores=NS),
    scratch_types=(
        pltpu.VMEM((CHUNK,), jnp.int32),
        pltpu.VMEM((CHUNK,), jnp.int32),
        pltpu.VMEM((NNEGC,), jnp.int32),
        pltpu.VMEM((CHUNK, DIM // 2), jnp.int32),
        pltpu.VMEM((CHUNK, DIM // 2), jnp.int32),
        pltpu.VMEM((NNEGC, DIM // 2), jnp.int32),
        pltpu.VMEM((BPW * LANES,), jnp.float32),
        pltpu.VMEM((BPW * LANES,), jnp.float32),
        pltpu.SemaphoreType.DMA,
    ),
    compiler_params=pltpu.CompilerParams(use_tc_tiling_on_sc=False),
  )


def _finish_body(spos_ref, sneg_ref, out_ref):
    il = lax.broadcasted_iota(jnp.int32, (128, 8), 0)
    ig = lax.broadcasted_iota(jnp.int32, (128, 8), 1)
    mask = (il // LANES == ig).astype(jnp.float32)
    sp = jnp.dot(spos_ref[...], mask, preferred_element_type=jnp.float32)
    sn = -jnp.dot(sneg_ref[...], mask, preferred_element_type=jnp.float32)

    def logsig(x):
        return jnp.minimum(x, 0.0) - jnp.log1p(jnp.exp(-jnp.abs(x)))

    loss = logsig(sp) + logsig(sn)
    out_ref[0, 0] = -jnp.sum(loss) / BATCH


_finish = pl.pallas_call(
    _finish_body,
    out_shape=jax.ShapeDtypeStruct((1, 1), jnp.float32),
    out_specs=pl.BlockSpec(memory_space=pltpu.SMEM),
)


@jax.jit
def kernel(u_pos, v_pos, v_neg, batch_size, U_emb, V_emb):
    del batch_size
    upos = u_pos.reshape(BATCH)
    vpos = v_pos.reshape(BATCH)
    vnegf = v_neg.reshape(NW * NCHUNK, NNEGC)
    U1 = _pack(U_emb.T).reshape(VOCAB, DIM // 2)
    V1 = _pack(V_emb.T).reshape(VOCAB, DIM // 2)
    spos, sneg = _sc_scores()(U1, V1, upos, vpos, vnegf)
    out = _finish(spos.reshape(BATCH * LANES // 128, 128),
                  sneg.reshape(BATCH * LANES // 128, 128))
    return out[0, 0]


# speedup vs baseline: 1.2899x; 1.2899x over previous
import functools

import jax
import jax.numpy as jnp
from jax import lax
from jax.experimental import pallas as pl
from jax.experimental.pallas import tpu as pltpu
from jax.experimental.pallas import tpu_sc as plsc

NC = 2
NS = 16
LANES = 16
NW = NC * NS

VOCAB = 1000000
BATCH = 16384
DIM = 64
NNEG = 20
KD = DIM // LANES

BPW = BATCH // NW
CHUNK = 64
NCHUNK = BPW // CHUNK
SLEN = 128
NNEGC = CHUNK * NNEG
NSTREAM = NNEGC // SLEN

PCOLS = 32000
NFULL = 31
TCOLS = VOCAB - NFULL * PCOLS
MAIN_END = NFULL * PCOLS
PGRID = NFULL + 1


def _pack_body(xt_hbm, out_hbm, buf0, buf1, ob0, ob1, tbuf, tob,
               si0, si1, so0, so1):
    i = pl.program_id(0)

    def start_in(blk, buf, sem):
        pltpu.make_async_copy(
            xt_hbm.at[:, pl.ds(blk * PCOLS, PCOLS)], buf, sem).start()

    even = lax.rem(i, 2) == 0

    @pl.when(i == 0)
    def _():
        start_in(0, buf0, si0)

    @pl.when(jnp.logical_and(i + 1 < NFULL, even))
    def _():
        start_in(i + 1, buf1, si1)

    @pl.when(jnp.logical_and(i + 1 < NFULL, jnp.logical_not(even)))
    def _():
        start_in(i + 1, buf0, si0)

    def pair_rows(x, n):
        u = lax.bitcast_convert_type(x, jnp.uint32)
        r = (u + jnp.uint32(0x7FFF) + ((u >> 16) & jnp.uint32(1))) >> 16
        lo = lax.slice(r, (0, 0), (DIM // 2, n))
        hi = lax.slice(r, (DIM // 2, 0), (DIM, n))
        packed = lax.bitcast_convert_type(lo | (hi << 16), jnp.int32)
        t = jnp.transpose(packed)
        q = n // 4
        return jnp.concatenate(
            [lax.slice(t, (h * q, 0), ((h + 1) * q, DIM // 2))
             for h in range(4)], axis=1)

    def do_block(buf, ob, sin, sout):
        pltpu.make_async_copy(xt_hbm.at[:, pl.ds(0, PCOLS)], buf, sin).wait()
        ob[...] = pair_rows(buf[...], PCOLS)
        pltpu.make_async_copy(
            ob, out_hbm.at[pl.ds(i * (PCOLS // 4), PCOLS // 4), :],
            sout).start()

    @pl.when(jnp.logical_and(i < NFULL, even))
    def _():
        @pl.when(i >= 2)
        def _():
            pltpu.make_async_copy(
                ob0, out_hbm.at[pl.ds(0, PCOLS // 4), :], so0).wait()
        do_block(buf0, ob0, si0, so0)

    @pl.when(jnp.logical_and(i < NFULL, jnp.logical_not(even)))
    def _():
        @pl.when(i >= 3)
        def _():
            pltpu.make_async_copy(
                ob1, out_hbm.at[pl.ds(0, PCOLS // 4), :], so1).wait()
        do_block(buf1, ob1, si1, so1)

    @pl.when(i == NFULL)
    def _():
        pltpu.make_async_copy(
            xt_hbm.at[:, pl.ds(NFULL * PCOLS, TCOLS)], tbuf, si0).start()
        pltpu.make_async_copy(
            xt_hbm.at[:, pl.ds(NFULL * PCOLS, TCOLS)], tbuf, si0).wait()
        tob[...] = pair_rows(tbuf[...], TCOLS)
        pltpu.make_async_copy(
            tob, out_hbm.at[pl.ds(NFULL * PCOLS // 4, TCOLS // 4), :],
            so0).start()
        pltpu.make_async_copy(
            ob1, out_hbm.at[pl.ds(0, PCOLS // 4), :], so1).wait()
        pltpu.make_async_copy(
            ob0, out_hbm.at[pl.ds(0, PCOLS // 4), :], so0).wait()
        pltpu.make_async_copy(
            tob, out_hbm.at[pl.ds(0, TCOLS // 4), :], so0).wait()


_pack = pl.pallas_call(
    _pack_body,
    grid=(PGRID,),
    in_specs=[pl.BlockSpec(memory_space=pl.ANY)],
    out_specs=pl.BlockSpec(memory_space=pl.ANY),
    out_shape=jax.ShapeDtypeStruct((VOCAB // 4, 2 * DIM), jnp.int32),
    compiler_params=pltpu.CompilerParams(fuse_transposed_lhs_in_matmul=True),
    scratch_shapes=[pltpu.VMEM((DIM, PCOLS), jnp.float32),
                    pltpu.VMEM((DIM, PCOLS), jnp.float32),
                    pltpu.VMEM((PCOLS // 4, 2 * DIM), jnp.int32),
                    pltpu.VMEM((PCOLS // 4, 2 * DIM), jnp.int32),
                    pltpu.VMEM((DIM, TCOLS), jnp.float32),
                    pltpu.VMEM((TCOLS // 4, 2 * DIM), jnp.int32),
                    pltpu.SemaphoreType.DMA,
                    pltpu.SemaphoreType.DMA,
                    pltpu.SemaphoreType.DMA,
                    pltpu.SemaphoreType.DMA],
)


def _sc_scores_body(Uemb, Vemb, upos, vpos, vnegf,
                    spos_out, sneg_out,
                    uidx, vidx, negidx, urows, vrows, negrows,
                    spos_acc, sneg_acc, sem):
    cid = lax.axis_index("c")
    sid = lax.axis_index("s")
    wid = cid * NS + sid
    wbase = wid * BPW

    def to_gidx(x):
        xt = x - MAIN_END
        blk = ((x >> 8) * 33555) >> 22
        rem = x - blk * PCOLS
        q = PCOLS // 4
        gm = blk * PCOLS + 4 * rem
        for t in (1, 2, 3):
            gm = gm - jnp.where(rem >= t * q, 4 * q - 1, 0)
        tq = TCOLS // 4
        gt = MAIN_END + 4 * xt
        for t in (1, 2, 3):
            gt = gt - jnp.where(xt >= t * tq, 4 * tq - 1, 0)
        return jnp.where(x >= MAIN_END, gt, gm)

    def chunk_body(ch, _):
        base = wbase + ch * CHUNK
        gchunk = wid * NCHUNK + ch
        pltpu.sync_copy(upos.at[pl.ds(base, CHUNK)], uidx)
        pltpu.sync_copy(vpos.at[pl.ds(base, CHUNK)], vidx)
        pltpu.sync_copy(vnegf.at[gchunk], negidx)
        for t in range(CHUNK // LANES):
            sl = pl.ds(t * LANES, LANES)
            uidx[sl] = to_gidx(uidx[sl])
            vidx[sl] = to_gidx(vidx[sl])
        for t in range(NNEGC // LANES):
            sl = pl.ds(t * LANES, LANES)
            negidx[sl] = to_gidx(negidx[sl])
        cps = [pltpu.async_copy(Uemb.at[uidx], urows, sem),
               pltpu.async_copy(Vemb.at[vidx], vrows, sem)]
        for s in range(NSTREAM):
            cps.append(pltpu.async_copy(
                Vemb.at[negidx.at[pl.ds(s * SLEN, SLEN)]],
                negrows.at[pl.ds(s * SLEN, SLEN), :], sem))
        for cp in cps:
            cp.wait()

        def load_row(ref, r):
            out = []
            for h in range(2):
                iv = ref[r, pl.ds(h * LANES, LANES)]
                lo = lax.bitcast_convert_type(iv << 16, jnp.float32)
                hi = lax.bitcast_convert_type(iv & jnp.int32(-65536),
                                              jnp.float32)
                out += [lo, hi]
            return out

        def item_body(b, _):
            u = load_row(urows, b)
            v = load_row(vrows, b)
            p = u[0] * v[0]
            for k in range(1, KD):
                p = p + u[k] * v[k]
            r0 = b * NNEG
            acc = load_row(negrows, r0)
            for j in range(1, NNEG):
                n = load_row(negrows, r0 + j)
                for k in range(KD):
                    acc[k] = acc[k] + n[k]
            q = acc[0] * u[0]
            for k in range(1, KD):
                q = q + acc[k] * u[k]
            off = (ch * CHUNK + b) * LANES
            spos_acc[pl.ds(off, LANES)] = p
            sneg_acc[pl.ds(off, LANES)] = q
            return 0

        lax.fori_loop(0, CHUNK, item_body, 0)
        return 0

    lax.fori_loop(0, NCHUNK, chunk_body, 0)
    pltpu.sync_copy(spos_acc, spos_out.at[pl.ds(wbase * LANES, BPW * LANES)])
    pltpu.sync_copy(sneg_acc, sneg_out.at[pl.ds(wbase * LANES, BPW * LANES)])


@functools.cache
def _sc_scores():
  return pl.kernel(
    _sc_scores_body,
    out_type=(jax.ShapeDtypeStruct((BATCH * LANES,), jnp.float32),
              jax.ShapeDtypeStruct((BATCH * LANES,), jnp.float32)),
    mesh=plsc.VectorSubcoreMesh(core_axis_name="c", subcore_axis_name="s",
                                num_cores=NC, num_subcores=NS),
    scratch_types=(
        pltpu.VMEM((CHUNK,), jnp.int32),
        pltpu.VMEM((CHUNK,), jnp.int32),
        pltpu.VMEM((NNEGC,), jnp.int32),
        pltpu.VMEM((CHUNK, DIM // 2), jnp.int32),
        pltpu.VMEM((CHUNK, DIM // 2), jnp.int32),
        pltpu.VMEM((NNEGC, DIM // 2), jnp.int32),
        pltpu.VMEM((BPW * LANES,), jnp.float32),
        pltpu.VMEM((BPW * LANES,), jnp.float32),
        pltpu.SemaphoreType.DMA,
    ),
    compiler_params=pltpu.CompilerParams(use_tc_tiling_on_sc=False),
  )


def _finish_body(spos_ref, sneg_ref, out_ref):
    il = lax.broadcasted_iota(jnp.int32, (128, 8), 0)
    ig = lax.broadcasted_iota(jnp.int32, (128, 8), 1)
    mask = (il // LANES == ig).astype(jnp.float32)
    sp = jnp.dot(spos_ref[...], mask, preferred_element_type=jnp.float32)
    sn = -jnp.dot(sneg_ref[...], mask, preferred_element_type=jnp.float32)

    def logsig(x):
        return jnp.minimum(x, 0.0) - jnp.log1p(jnp.exp(-jnp.abs(x)))

    loss = logsig(sp) + logsig(sn)
    out_ref[0, 0] = -jnp.sum(loss) / BATCH


_finish = pl.pallas_call(
    _finish_body,
    out_shape=jax.ShapeDtypeStruct((1, 1), jnp.float32),
    out_specs=pl.BlockSpec(memory_space=pltpu.SMEM),
)


@jax.jit
def kernel(u_pos, v_pos, v_neg, batch_size, U_emb, V_emb):
    del batch_size
    upos = u_pos.reshape(BATCH)
    vpos = v_pos.reshape(BATCH)
    vnegf = v_neg.reshape(NW * NCHUNK, NNEGC)
    U1 = _pack(U_emb.T).reshape(VOCAB, DIM // 2)
    V1 = _pack(V_emb.T).reshape(VOCAB, DIM // 2)
    spos, sneg = _sc_scores()(U1, V1, upos, vpos, vnegf)
    out = _finish(spos.reshape(BATCH * LANES // 128, 128),
                  sneg.reshape(BATCH * LANES // 128, 128))
    return out[0, 0]


# speedup vs baseline: 1.6208x; 1.2565x over previous
import functools

import jax
import jax.numpy as jnp
from jax import lax
from jax.experimental import pallas as pl
from jax.experimental.pallas import tpu as pltpu
from jax.experimental.pallas import tpu_sc as plsc

NC = 2
NS = 16
LANES = 16
NW = NC * NS

VOCAB = 1000000
BATCH = 16384
DIM = 64
NNEG = 20
KD = DIM // LANES

BPW = BATCH // NW
CHUNK = 64
NCHUNK = BPW // CHUNK
SLEN = 128
NNEGC = CHUNK * NNEG
NSTREAM = NNEGC // SLEN

PCOLS = 32000
NFULL = 31
TCOLS = VOCAB - NFULL * PCOLS
MAIN_END = NFULL * PCOLS
PGRID = NFULL + 1


def _pack_body(xt_hbm, out_hbm, buf0, buf1, ob0, ob1, tbuf, tob,
               si0, si1, so0, so1):
    i = pl.program_id(0)

    def start_in(blk, buf, sem):
        pltpu.make_async_copy(
            xt_hbm.at[:, pl.ds(blk * PCOLS, PCOLS)], buf, sem).start()

    even = lax.rem(i, 2) == 0

    @pl.when(i == 0)
    def _():
        start_in(0, buf0, si0)

    @pl.when(jnp.logical_and(i + 1 < NFULL, even))
    def _():
        start_in(i + 1, buf1, si1)

    @pl.when(jnp.logical_and(i + 1 < NFULL, jnp.logical_not(even)))
    def _():
        start_in(i + 1, buf0, si0)

    def pair_rows(x, n):
        q = n // 2
        xa = lax.slice(x, (0, 0), (DIM, q))
        xb = lax.slice(x, (0, q), (DIM, n))
        ri = lax.broadcasted_iota(jnp.int32, (DIM, 2 * DIM), 0)
        ci = lax.broadcasted_iota(jnp.int32, (DIM, 2 * DIM), 1)
        w1 = (ci == ri).astype(jnp.float32)
        w2 = (ci == ri + DIM).astype(jnp.float32)
        return (jax.lax.dot(xa.T, w1, preferred_element_type=jnp.float32) +
                jax.lax.dot(xb.T, w2, preferred_element_type=jnp.float32))

    def do_block(buf, ob, sin, sout):
        pltpu.make_async_copy(xt_hbm.at[:, pl.ds(0, PCOLS)], buf, sin).wait()
        ob[...] = pair_rows(buf[...], PCOLS)
        pltpu.make_async_copy(
            ob, out_hbm.at[pl.ds(i * (PCOLS // 2), PCOLS // 2), :],
            sout).start()

    @pl.when(jnp.logical_and(i < NFULL, even))
    def _():
        @pl.when(i >= 2)
        def _():
            pltpu.make_async_copy(
                ob0, out_hbm.at[pl.ds(0, PCOLS // 2), :], so0).wait()
        do_block(buf0, ob0, si0, so0)

    @pl.when(jnp.logical_and(i < NFULL, jnp.logical_not(even)))
    def _():
        @pl.when(i >= 3)
        def _():
            pltpu.make_async_copy(
                ob1, out_hbm.at[pl.ds(0, PCOLS // 2), :], so1).wait()
        do_block(buf1, ob1, si1, so1)

    @pl.when(i == NFULL)
    def _():
        pltpu.make_async_copy(
            xt_hbm.at[:, pl.ds(NFULL * PCOLS, TCOLS)], tbuf, si0).start()
        pltpu.make_async_copy(
            xt_hbm.at[:, pl.ds(NFULL * PCOLS, TCOLS)], tbuf, si0).wait()
        tob[...] = pair_rows(tbuf[...], TCOLS)
        pltpu.make_async_copy(
            tob, out_hbm.at[pl.ds(NFULL * PCOLS // 2, TCOLS // 2), :],
            so0).start()
        pltpu.make_async_copy(
            ob1, out_hbm.at[pl.ds(0, PCOLS // 2), :], so1).wait()
        pltpu.make_async_copy(
            ob0, out_hbm.at[pl.ds(0, PCOLS // 2), :], so0).wait()
        pltpu.make_async_copy(
            tob, out_hbm.at[pl.ds(0, TCOLS // 2), :], so0).wait()


_pack = pl.pallas_call(
    _pack_body,
    grid=(PGRID,),
    in_specs=[pl.BlockSpec(memory_space=pl.ANY)],
    out_specs=pl.BlockSpec(memory_space=pl.ANY),
    out_shape=jax.ShapeDtypeStruct((VOCAB // 2, 2 * DIM), jnp.float32),
    compiler_params=pltpu.CompilerParams(fuse_transposed_lhs_in_matmul=True),
    scratch_shapes=[pltpu.VMEM((DIM, PCOLS), jnp.float32),
                    pltpu.VMEM((DIM, PCOLS), jnp.float32),
                    pltpu.VMEM((PCOLS // 2, 2 * DIM), jnp.float32),
                    pltpu.VMEM((PCOLS // 2, 2 * DIM), jnp.float32),
                    pltpu.VMEM((DIM, TCOLS), jnp.float32),
                    pltpu.VMEM((TCOLS // 2, 2 * DIM), jnp.float32),
                    pltpu.SemaphoreType.DMA,
                    pltpu.SemaphoreType.DMA,
                    pltpu.SemaphoreType.DMA,
                    pltpu.SemaphoreType.DMA],
)


def _sc_scores_body(Uemb, Vemb, upos, vpos, vnegf,
                    spos_out, sneg_out,
                    uidx, vidx, negidx, urows, vrows, negrows,
                    spos_acc, sneg_acc, sem):
    cid = lax.axis_index("c")
    sid = lax.axis_index("s")
    wid = cid * NS + sid
    wbase = wid * BPW

    def to_gidx(x):
        xt = x - MAIN_END
        blk = ((x >> 8) * 33555) >> 22
        rem = x - blk * PCOLS
        gm = blk * PCOLS + 2 * rem - jnp.where(rem >= PCOLS // 2,
                                               PCOLS - 1, 0)
        gt = MAIN_END + 2 * xt - jnp.where(xt >= TCOLS // 2, TCOLS - 1, 0)
        return jnp.where(x >= MAIN_END, gt, gm)

    def chunk_body(ch, _):
        base = wbase + ch * CHUNK
        gchunk = wid * NCHUNK + ch
        pltpu.sync_copy(upos.at[pl.ds(base, CHUNK)], uidx)
        pltpu.sync_copy(vpos.at[pl.ds(base, CHUNK)], vidx)
        pltpu.sync_copy(vnegf.at[gchunk], negidx)
        for t in range(CHUNK // LANES):
            sl = pl.ds(t * LANES, LANES)
            uidx[sl] = to_gidx(uidx[sl])
            vidx[sl] = to_gidx(vidx[sl])
        for t in range(NNEGC // LANES):
            sl = pl.ds(t * LANES, LANES)
            negidx[sl] = to_gidx(negidx[sl])
        cps = [pltpu.async_copy(Uemb.at[uidx], urows, sem),
               pltpu.async_copy(Vemb.at[vidx], vrows, sem)]
        for s in range(NSTREAM):
            cps.append(pltpu.async_copy(
                Vemb.at[negidx.at[pl.ds(s * SLEN, SLEN)]],
                negrows.at[pl.ds(s * SLEN, SLEN), :], sem))
        for cp in cps:
            cp.wait()

        def item_body(b, _):
            u = [urows[b, pl.ds(k * LANES, LANES)] for k in range(KD)]
            v = [vrows[b, pl.ds(k * LANES, LANES)] for k in range(KD)]
            p = u[0] * v[0]
            for k in range(1, KD):
                p = p + u[k] * v[k]
            r0 = b * NNEG
            acc = [negrows[r0, pl.ds(k * LANES, LANES)] for k in range(KD)]
            for j in range(1, NNEG):
                for k in range(KD):
                    acc[k] = acc[k] + negrows[r0 + j, pl.ds(k * LANES, LANES)]
            q = acc[0] * u[0]
            for k in range(1, KD):
                q = q + acc[k] * u[k]
            off = (ch * CHUNK + b) * LANES
            spos_acc[pl.ds(off, LANES)] = p
            sneg_acc[pl.ds(off, LANES)] = q
            return 0

        lax.fori_loop(0, CHUNK, item_body, 0)
        return 0

    lax.fori_loop(0, NCHUNK, chunk_body, 0)
    pltpu.sync_copy(spos_acc, spos_out.at[pl.ds(wbase * LANES, BPW * LANES)])
    pltpu.sync_copy(sneg_acc, sneg_out.at[pl.ds(wbase * LANES, BPW * LANES)])


@functools.cache
def _sc_scores():
  return pl.kernel(
    _sc_scores_body,
    out_type=(jax.ShapeDtypeStruct((BATCH * LANES,), jnp.float32),
              jax.ShapeDtypeStruct((BATCH * LANES,), jnp.float32)),
    mesh=plsc.VectorSubcoreMesh(core_axis_name="c", subcore_axis_name="s",
                                num_cores=NC, num_subcores=NS),
    scratch_types=(
        pltpu.VMEM((CHUNK,), jnp.int32),
        pltpu.VMEM((CHUNK,), jnp.int32),
        pltpu.VMEM((NNEGC,), jnp.int32),
        pltpu.VMEM((CHUNK, DIM), jnp.float32),
        pltpu.VMEM((CHUNK, DIM), jnp.float32),
        pltpu.VMEM((NNEGC, DIM), jnp.float32),
        pltpu.VMEM((BPW * LANES,), jnp.float32),
        pltpu.VMEM((BPW * LANES,), jnp.float32),
        pltpu.SemaphoreType.DMA,
    ),
    compiler_params=pltpu.CompilerParams(use_tc_tiling_on_sc=False),
  )


def _finish_body(spos_ref, sneg_ref, out_ref):
    il = lax.broadcasted_iota(jnp.int32, (128, 8), 0)
    ig = lax.broadcasted_iota(jnp.int32, (128, 8), 1)
    mask = (il // LANES == ig).astype(jnp.float32)
    sp = jnp.dot(spos_ref[...], mask, preferred_element_type=jnp.float32)
    sn = -jnp.dot(sneg_ref[...], mask, preferred_element_type=jnp.float32)

    def logsig(x):
        return jnp.minimum(x, 0.0) - jnp.log1p(jnp.exp(-jnp.abs(x)))

    loss = logsig(sp) + logsig(sn)
    out_ref[0, 0] = -jnp.sum(loss) / BATCH


_finish = pl.pallas_call(
    _finish_body,
    out_shape=jax.ShapeDtypeStruct((1, 1), jnp.float32),
    out_specs=pl.BlockSpec(memory_space=pltpu.SMEM),
)


@jax.jit
def kernel(u_pos, v_pos, v_neg, batch_size, U_emb, V_emb):
    del batch_size
    upos = u_pos.reshape(BATCH)
    vpos = v_pos.reshape(BATCH)
    vnegf = v_neg.reshape(NW * NCHUNK, NNEGC)
    U1 = _pack(U_emb.T).reshape(VOCAB, DIM)
    V1 = _pack(V_emb.T).reshape(VOCAB, DIM)
    spos, sneg = _sc_scores()(U1, V1, upos, vpos, vnegf)
    out = _finish(spos.reshape(BATCH * LANES // 128, 128),
                  sneg.reshape(BATCH * LANES // 128, 128))
    return out[0, 0]
